# baseline (speedup 1.0000x reference)
import functools

import jax
import jax.numpy as jnp
from jax import lax
from jax.experimental import pallas as pl
from jax.experimental.pallas import tpu as pltpu
from jax.experimental.pallas import tpu_sc as plsc

EMBEDDING_DIM = 64
NUM_EMBEDDINGS = 1024
ROWS_PER_TILE = 512

_NUM_CORES = 2
_NUM_SUBCORES = 16
_NUM_WORKERS = _NUM_CORES * _NUM_SUBCORES
_IDX_CHUNK = 96


def _argmin_body(x_ref, w_ref, xsq_ref, wsq_ref, idx_ref, table_ref):
    xb = x_ref[...]
    w = w_ref[...]
    xw = jnp.dot(xb, w)
    distances = xsq_ref[...] - 2.0 * xw + wsq_ref[...]
    idx_ref[...] = jnp.argmin(distances, axis=1).astype(jnp.int32)

    @pl.when(pl.program_id(0) == 0)
    def _():
        table_ref[...] = jnp.swapaxes(w, 0, 1)


def _encode_indices(flat, w):
    n = flat.shape[0]
    grid = n // ROWS_PER_TILE
    xsq = jnp.sum(flat ** 2, axis=1, keepdims=True)
    wsq = jnp.sum(w ** 2, axis=0, keepdims=True)
    return pl.pallas_call(
        _argmin_body,
        grid=(grid,),
        in_specs=[
            pl.BlockSpec((ROWS_PER_TILE, EMBEDDING_DIM), lambda i: (i, 0)),
            pl.BlockSpec((EMBEDDING_DIM, NUM_EMBEDDINGS), lambda i: (0, 0)),
            pl.BlockSpec((ROWS_PER_TILE, 1), lambda i: (i, 0)),
            pl.BlockSpec((1, NUM_EMBEDDINGS), lambda i: (0, 0)),
        ],
        out_specs=[
            pl.BlockSpec((ROWS_PER_TILE,), lambda i: (i,)),
            pl.BlockSpec((NUM_EMBEDDINGS, EMBEDDING_DIM), lambda i: (0, 0)),
        ],
        out_shape=[
            jax.ShapeDtypeStruct((n,), jnp.int32),
            jax.ShapeDtypeStruct((NUM_EMBEDDINGS, EMBEDDING_DIM), jnp.float32),
        ],
        compiler_params=pltpu.CompilerParams(
            dimension_semantics=("arbitrary",)),
    )(flat, w, xsq, wsq)


def _make_sc_gather(n):
    b_per_w = n // _NUM_WORKERS
    n_chunks = b_per_w // _IDX_CHUNK
    mesh = plsc.VectorSubcoreMesh(core_axis_name="c", subcore_axis_name="s")

    @functools.partial(
        pl.kernel, mesh=mesh,
        out_type=jax.ShapeDtypeStruct((n, EMBEDDING_DIM), jnp.float32),
        scratch_types=[
            pltpu.VMEM((b_per_w,), jnp.int32),
            pltpu.VMEM((b_per_w, EMBEDDING_DIM), jnp.float32),
            pltpu.SemaphoreType.DMA,
        ],
        compiler_params=pltpu.CompilerParams(use_tc_tiling_on_sc=False),
    )
    def gather_kernel(table_hbm, idx_hbm, out_hbm, idx_v, rows_v, sem):
        wid = lax.axis_index("s") * _NUM_CORES + lax.axis_index("c")
        base = wid * b_per_w
        pltpu.sync_copy(idx_hbm.at[pl.ds(base, b_per_w)], idx_v)
        copies = []
        for j in range(n_chunks):
            off = j * _IDX_CHUNK
            copies.append(pltpu.async_copy(
                table_hbm.at[idx_v.at[pl.ds(off, _IDX_CHUNK)]],
                rows_v.at[pl.ds(off, _IDX_CHUNK)],
                sem))
        for c in copies:
            c.wait()
        pltpu.sync_copy(rows_v, out_hbm.at[pl.ds(base, b_per_w)])

    return gather_kernel


def kernel(x, w):
    flat = jnp.reshape(x, (-1, EMBEDDING_DIM))
    n = flat.shape[0]
    indices, table = _encode_indices(flat, w)
    quantized = _make_sc_gather(n)(table, indices)
    return jnp.reshape(quantized, x.shape)
